# baseline (device time: 36399 ns/iter reference)
import jax
import jax.numpy as jnp
from jax import lax
from jax.experimental import pallas as pl
from jax.experimental.pallas import tpu as pltpu

ROWS = 256
HALF = 4096
NC = 8
CHUNK = HALF // NC


def kernel(x, W):
    def body(x_ref, w_ref, out_ref, send_buf, recv_buf, send_sems, recv_sems):
        my_x = lax.axis_index("x")
        my_y = lax.axis_index("y")
        my_z = lax.axis_index("z")
        partner = (my_x, 1 - my_y, my_z)

        barrier_sem = pltpu.get_barrier_semaphore()
        pl.semaphore_signal(
            barrier_sem, inc=1, device_id=partner,
            device_id_type=pl.DeviceIdType.MESH,
        )
        pl.semaphore_wait(barrier_sem, 1)

        xl = x_ref[...].astype(jnp.bfloat16)

        def chunk_rdma(k):
            cs = pl.ds(k * CHUNK, CHUNK)
            return pltpu.make_async_remote_copy(
                src_ref=send_buf.at[:, cs],
                dst_ref=recv_buf.at[:, cs],
                send_sem=send_sems.at[k],
                recv_sem=recv_sems.at[k],
                device_id=partner,
                device_id_type=pl.DeviceIdType.MESH,
            )

        s_loc = jnp.zeros((ROWS, 1), jnp.float32)
        for k in range(NC):
            cs = pl.ds(k * CHUNK, CHUNK)
            wk = w_ref[:, cs].astype(jnp.bfloat16)
            ek = jnp.exp(jnp.dot(xl, wk, preferred_element_type=jnp.float32))
            send_buf[:, cs] = ek.astype(jnp.bfloat16)
            chunk_rdma(k).start()
            s_loc = s_loc + jnp.sum(ek, axis=1, keepdims=True)

        s_rem = jnp.zeros((ROWS, 1), jnp.float32)
        for k in range(NC):
            chunk_rdma(k).wait_recv()
            cs = pl.ds(k * CHUNK, CHUNK)
            ck = recv_buf[:, cs].astype(jnp.float32)
            s_rem = s_rem + jnp.sum(ck, axis=1, keepdims=True)

        inv = 1.0 / (s_loc + s_rem)
        p_loc = send_buf[...].astype(jnp.float32) * inv
        p_rem = recv_buf[...].astype(jnp.float32) * inv

        @pl.when(my_y == 0)
        def _():
            out_ref[:, :HALF] = p_loc
            out_ref[:, HALF:] = p_rem

        @pl.when(my_y == 1)
        def _():
            out_ref[:, :HALF] = p_rem
            out_ref[:, HALF:] = p_loc

        for k in range(NC):
            chunk_rdma(k).wait_send()

    return pl.pallas_call(
        body,
        out_shape=jax.ShapeDtypeStruct((ROWS, 2 * HALF), jnp.float32),
        in_specs=[
            pl.BlockSpec(memory_space=pltpu.VMEM),
            pl.BlockSpec(memory_space=pltpu.VMEM),
        ],
        out_specs=pl.BlockSpec(memory_space=pltpu.VMEM),
        scratch_shapes=[
            pltpu.VMEM((ROWS, HALF), jnp.bfloat16),
            pltpu.VMEM((ROWS, HALF), jnp.bfloat16),
            pltpu.SemaphoreType.DMA((NC,)),
            pltpu.SemaphoreType.DMA((NC,)),
        ],
        compiler_params=pltpu.CompilerParams(collective_id=0),
    )(x, W)


# device time: 34845 ns/iter; 1.0446x vs baseline; 1.0446x over previous
import jax
import jax.numpy as jnp
from jax import lax
from jax.experimental import pallas as pl
from jax.experimental.pallas import tpu as pltpu

ROWS = 256
HALF = 4096
NC = 8
CHUNK = HALF // NC


def kernel(x, W):
    def body(x_ref, w_ref, out_ref, send_buf, recv_buf, send_sems, recv_sems):
        my_x = lax.axis_index("x")
        my_y = lax.axis_index("y")
        my_z = lax.axis_index("z")
        partner = (my_x, 1 - my_y, my_z)

        barrier_sem = pltpu.get_barrier_semaphore()
        pl.semaphore_signal(
            barrier_sem, inc=1, device_id=partner,
            device_id_type=pl.DeviceIdType.MESH,
        )
        pl.semaphore_wait(barrier_sem, 1)

        xl = x_ref[...].astype(jnp.bfloat16)

        def chunk_rdma(k):
            cs = pl.ds(k * CHUNK, CHUNK)
            return pltpu.make_async_remote_copy(
                src_ref=send_buf.at[:, cs],
                dst_ref=recv_buf.at[:, cs],
                send_sem=send_sems.at[k],
                recv_sem=recv_sems.at[k],
                device_id=partner,
                device_id_type=pl.DeviceIdType.MESH,
            )

        s_loc = jnp.zeros((ROWS, 1), jnp.float32)
        for k in range(NC):
            cs = pl.ds(k * CHUNK, CHUNK)
            wk = w_ref[:, cs].astype(jnp.bfloat16)
            ek = jnp.exp(jnp.dot(xl, wk, preferred_element_type=jnp.float32))
            send_buf[:, cs] = ek.astype(jnp.bfloat16)
            chunk_rdma(k).start()
            s_loc = s_loc + jnp.sum(ek, axis=1, keepdims=True)

        s_rem = jnp.zeros((ROWS, 1), jnp.float32)
        for k in range(NC):
            chunk_rdma(k).wait_recv()
            cs = pl.ds(k * CHUNK, CHUNK)
            ck = recv_buf[:, cs].astype(jnp.float32)
            s_rem = s_rem + jnp.sum(ck, axis=1, keepdims=True)

        inv = 1.0 / (s_loc + s_rem)
        p_loc = (send_buf[...].astype(jnp.float32) * inv).astype(jnp.bfloat16)
        p_rem = (recv_buf[...].astype(jnp.float32) * inv).astype(jnp.bfloat16)

        @pl.when(my_y == 0)
        def _():
            out_ref[:, :HALF] = p_loc
            out_ref[:, HALF:] = p_rem

        @pl.when(my_y == 1)
        def _():
            out_ref[:, :HALF] = p_rem
            out_ref[:, HALF:] = p_loc

        for k in range(NC):
            chunk_rdma(k).wait_send()

    return pl.pallas_call(
        body,
        out_shape=jax.ShapeDtypeStruct((ROWS, 2 * HALF), jnp.bfloat16),
        in_specs=[
            pl.BlockSpec(memory_space=pltpu.VMEM),
            pl.BlockSpec(memory_space=pltpu.VMEM),
        ],
        out_specs=pl.BlockSpec(memory_space=pltpu.VMEM),
        scratch_shapes=[
            pltpu.VMEM((ROWS, HALF), jnp.bfloat16),
            pltpu.VMEM((ROWS, HALF), jnp.bfloat16),
            pltpu.SemaphoreType.DMA((NC,)),
            pltpu.SemaphoreType.DMA((NC,)),
        ],
        compiler_params=pltpu.CompilerParams(collective_id=0),
    )(x, W)
